# baseline (speedup 1.0000x reference)
import jax
import jax.numpy as jnp
from jax.experimental import pallas as pl
from jax.experimental.pallas import tpu as pltpu

_B, _S, _D, _E, _K = 2, 2048, 1024, 8, 2
_H = 4 * _D
_N = _B * _S
_T = 512
_NC = 2
_NTC = _N // (_NC * _T)
_HC = 1024


def _moe_body(x_ref, wg_ref, w1_ref, b1_ref, w2_ref, b2_ref,
              out_ref, mask_sc, h_sc):
    c = pl.program_id(0)
    e = pl.program_id(2)

    x_bf = x_ref[...].astype(jnp.bfloat16)

    @pl.when(e == 0)
    def _():
        g = jnp.dot(x_bf, wg_ref[...].astype(jnp.bfloat16),
                    preferred_element_type=jnp.float32)
        m1 = jnp.max(g, axis=1, keepdims=True)
        at_max = g >= m1
        n_max = jnp.sum(at_max.astype(jnp.float32), axis=1, keepdims=True)
        m2 = jnp.max(jnp.where(at_max, -jnp.inf, g), axis=1, keepdims=True)
        kth = jnp.where(n_max >= _K, m1, m2)
        keep = g >= kth
        p = jnp.where(keep, jnp.exp(g - m1), 0.0)
        mask_sc[c] = p / jnp.sum(p, axis=1, keepdims=True) * (1.0 / _E)

    msel = mask_sc[c]
    lane = jax.lax.broadcasted_iota(jnp.int32, (_T, _E), 1)
    w_e = jnp.sum(jnp.where(lane == e, msel, 0.0), axis=1, keepdims=True)

    w1 = w1_ref[0]
    b1 = b1_ref[0]
    for i in range(_H // _HC):
        sl = slice(i * _HC, (i + 1) * _HC)
        hc = jnp.dot(x_bf, w1[:, sl], preferred_element_type=jnp.float32)
        hc = jnp.maximum(hc + b1[:, sl], 0.0)
        h_sc[c, :, sl] = hc.astype(jnp.bfloat16)

    y = jnp.dot(h_sc[c], w2_ref[0], preferred_element_type=jnp.float32)
    y = (y + b2_ref[0]) * w_e

    @pl.when(e == 0)
    def _():
        out_ref[...] = y

    @pl.when(e > 0)
    def _():
        out_ref[...] += y


def kernel(x, W_gate, W1, b1, W2, b2):
    xf = x.reshape(_N, _D)
    w1b = W1.astype(jnp.bfloat16)
    w2b = W2.astype(jnp.bfloat16)
    out = pl.pallas_call(
        _moe_body,
        grid=(_NC, _NTC, _E),
        in_specs=[
            pl.BlockSpec((_T, _D), lambda c, t, e: (c * _NTC + t, 0)),
            pl.BlockSpec((_D, _E), lambda c, t, e: (0, 0)),
            pl.BlockSpec((1, _D, _H), lambda c, t, e: (e, 0, 0)),
            pl.BlockSpec((1, 1, _H), lambda c, t, e: (e, 0, 0)),
            pl.BlockSpec((1, _H, _D), lambda c, t, e: (e, 0, 0)),
            pl.BlockSpec((1, 1, _D), lambda c, t, e: (e, 0, 0)),
        ],
        out_specs=pl.BlockSpec((_T, _D), lambda c, t, e: (c * _NTC + t, 0)),
        out_shape=jax.ShapeDtypeStruct((_N, _D), jnp.float32),
        scratch_shapes=[
            pltpu.VMEM((_NC, _T, _E), jnp.float32),
            pltpu.VMEM((_NC, _T, _H), jnp.bfloat16),
        ],
        compiler_params=pltpu.CompilerParams(
            dimension_semantics=("parallel", "arbitrary", "arbitrary"),
            vmem_limit_bytes=56 * 1024 * 1024,
        ),
        name="moe_fused",
    )(xf, W_gate, w1b, b1.reshape(_E, 1, _H), w2b, b2.reshape(_E, 1, _D))
    return out.reshape(_B, _S, _D)


# speedup vs baseline: 1.4225x; 1.4225x over previous
import jax
import jax.numpy as jnp
from jax.experimental import pallas as pl
from jax.experimental.pallas import tpu as pltpu

_B, _S, _D, _E, _K = 2, 2048, 1024, 8, 2
_H = 4 * _D
_N = _B * _S
_A = _N * _K
_BT = 256
_NB = _A // _BT + _E
_APAD = _NB * _BT
_NC = 2
_NBC = _NB // _NC
_TG = 1024
_TC = 512


def _gate_body(x_ref, wg_ref, mask_ref):
    g = jnp.dot(x_ref[...].astype(jnp.bfloat16),
                wg_ref[...].astype(jnp.bfloat16),
                preferred_element_type=jnp.float32)
    m1 = jnp.max(g, axis=1, keepdims=True)
    at_max = g >= m1
    n_max = jnp.sum(at_max.astype(jnp.float32), axis=1, keepdims=True)
    m2 = jnp.max(jnp.where(at_max, -jnp.inf, g), axis=1, keepdims=True)
    kth = jnp.where(n_max >= _K, m1, m2)
    keep = g >= kth
    p = jnp.where(keep, jnp.exp(g - m1), 0.0)
    mask_ref[...] = p / jnp.sum(p, axis=1, keepdims=True) * (1.0 / _E)


def _gate(xf, W_gate):
    nt = _N // (_NC * _TG)
    return pl.pallas_call(
        _gate_body,
        grid=(_NC, nt),
        in_specs=[
            pl.BlockSpec((_TG, _D), lambda c, t: (c * nt + t, 0)),
            pl.BlockSpec((_D, _E), lambda c, t: (0, 0)),
        ],
        out_specs=pl.BlockSpec((_TG, _E), lambda c, t: (c * nt + t, 0)),
        out_shape=jax.ShapeDtypeStruct((_N, _E), jnp.float32),
        compiler_params=pltpu.CompilerParams(
            dimension_semantics=("parallel", "arbitrary"),
        ),
        name="moe_gate",
    )(xf, W_gate)


def _ffn_body(eob_ref, tok_ref, x_hbm, w1_ref, b1_ref, w2_ref, b2_ref,
              ys_ref, xfull, xg, xgb, h_sc, sem):
    c = pl.program_id(0)
    t = pl.program_id(1)
    b = c * _NBC + t

    @pl.when(t == 0)
    def _():
        cp = pltpu.make_async_copy(x_hbm, xfull, sem)
        cp.start()
        cp.wait()

    @pl.when(b < eob_ref[_NB])
    def _():
        base = b * _BT
        for r in range(_BT):
            xg[c, r] = xfull[tok_ref[base + r], :]
        xgb[c] = xg[c].astype(jnp.bfloat16)

        x_bf = xgb[c]
        w1 = w1_ref[0]
        b1 = b1_ref[0]
        for i in range(2):
            sl = slice(i * (_H // 2), (i + 1) * (_H // 2))
            hc = jnp.dot(x_bf, w1[:, sl], preferred_element_type=jnp.float32)
            h_sc[c, :, sl] = jnp.maximum(hc + b1[:, sl], 0.0
                                         ).astype(jnp.bfloat16)
        y = jnp.dot(h_sc[c], w2_ref[0], preferred_element_type=jnp.float32)
        ys_ref[...] = y + b2_ref[0]


def _ffn(xf, w1b, b1, w2b, b2, eob, tok_sorted):
    grid_spec = pltpu.PrefetchScalarGridSpec(
        num_scalar_prefetch=2,
        grid=(_NC, _NBC),
        in_specs=[
            pl.BlockSpec(memory_space=pl.ANY),
            pl.BlockSpec((1, _D, _H),
                         lambda c, t, eob, tok: (eob[c * _NBC + t], 0, 0)),
            pl.BlockSpec((1, 1, _H),
                         lambda c, t, eob, tok: (eob[c * _NBC + t], 0, 0)),
            pl.BlockSpec((1, _H, _D),
                         lambda c, t, eob, tok: (eob[c * _NBC + t], 0, 0)),
            pl.BlockSpec((1, 1, _D),
                         lambda c, t, eob, tok: (eob[c * _NBC + t], 0, 0)),
        ],
        out_specs=pl.BlockSpec((_BT, _D),
                               lambda c, t, eob, tok: (c * _NBC + t, 0)),
        scratch_shapes=[
            pltpu.VMEM((_N, _D), jnp.float32),
            pltpu.VMEM((_NC, _BT, _D), jnp.float32),
            pltpu.VMEM((_NC, _BT, _D), jnp.bfloat16),
            pltpu.VMEM((_NC, _BT, _H), jnp.bfloat16),
            pltpu.SemaphoreType.DMA,
        ],
    )
    return pl.pallas_call(
        _ffn_body,
        grid_spec=grid_spec,
        out_shape=jax.ShapeDtypeStruct((_APAD, _D), jnp.float32),
        compiler_params=pltpu.CompilerParams(
            dimension_semantics=("parallel", "arbitrary"),
            vmem_limit_bytes=58 * 1024 * 1024,
        ),
        name="moe_ffn",
    )(eob, tok_sorted, xf, w1b, b1, w2b, b2)


def _combine_body(p0_ref, p1_ref, ys_hbm, w_ref, out_ref,
                  ysfull, y0, y1, sem):
    c = pl.program_id(0)
    t = pl.program_id(1)
    nt = _N // (_NC * _TC)
    base = (c * nt + t) * _TC

    @pl.when(t == 0)
    def _():
        cp = pltpu.make_async_copy(ys_hbm, ysfull, sem)
        cp.start()
        cp.wait()

    for r in range(_TC):
        y0[c, r] = ysfull[p0_ref[base + r], :]
        y1[c, r] = ysfull[p1_ref[base + r], :]

    w = w_ref[...]
    lane = jax.lax.broadcasted_iota(jnp.int32, (_TC, _E), 1)
    w0 = jnp.sum(jnp.where(lane == 0, w, 0.0), axis=1, keepdims=True)
    w1 = jnp.sum(jnp.where(lane == 1, w, 0.0), axis=1, keepdims=True)
    out_ref[...] = w0 * y0[c] + w1 * y1[c]


def _combine(ys, wpad, p0, p1):
    nt = _N // (_NC * _TC)
    grid_spec = pltpu.PrefetchScalarGridSpec(
        num_scalar_prefetch=2,
        grid=(_NC, nt),
        in_specs=[
            pl.BlockSpec(memory_space=pl.ANY),
            pl.BlockSpec((_TC, _E), lambda c, t, p0, p1: (c * nt + t, 0)),
        ],
        out_specs=pl.BlockSpec((_TC, _D), lambda c, t, p0, p1: (c * nt + t, 0)),
        scratch_shapes=[
            pltpu.VMEM((_APAD, _D), jnp.float32),
            pltpu.VMEM((_NC, _TC, _D), jnp.float32),
            pltpu.VMEM((_NC, _TC, _D), jnp.float32),
            pltpu.SemaphoreType.DMA,
        ],
    )
    return pl.pallas_call(
        _combine_body,
        grid_spec=grid_spec,
        out_shape=jax.ShapeDtypeStruct((_N, _D), jnp.float32),
        compiler_params=pltpu.CompilerParams(
            dimension_semantics=("parallel", "arbitrary"),
            vmem_limit_bytes=56 * 1024 * 1024,
        ),
        name="moe_combine",
    )(p0, p1, ys, wpad)


def kernel(x, W_gate, W1, b1, W2, b2):
    xf = x.reshape(_N, _D)
    w1b = W1.astype(jnp.bfloat16)
    w2b = W2.astype(jnp.bfloat16)

    mask = _gate(xf, W_gate)

    i32 = jnp.int32
    e1 = jnp.argmax(mask, axis=1).astype(i32)
    lane = jnp.arange(_E, dtype=i32)[None, :]
    m2v = jnp.where(lane == e1[:, None], -1.0, mask)
    e2 = jnp.argmax(m2v, axis=1).astype(i32)
    w1v = jnp.take_along_axis(mask, e1[:, None], axis=1)[:, 0]
    w2v = jnp.take_along_axis(mask, e2[:, None], axis=1)[:, 0]

    eflat = jnp.stack([e1, e2], axis=1).reshape(_A)
    tokf = jnp.repeat(jnp.arange(_N, dtype=i32), _K)
    onehot = (eflat[:, None] == lane).astype(i32)
    ranks = jnp.cumsum(onehot, axis=0) - 1
    rank_j = jnp.take_along_axis(ranks, eflat[:, None], axis=1)[:, 0]
    counts = jnp.sum(onehot, axis=0)
    nblk = (counts + _BT - 1) // _BT
    blk_end = jnp.cumsum(nblk).astype(i32)
    blk_start = jnp.concatenate([jnp.zeros(1, i32), blk_end[:-1]])
    dst = blk_start[eflat] * _BT + rank_j

    tok_sorted = jnp.zeros(_APAD, i32).at[dst].set(tokf)
    blks = jnp.arange(_NB, dtype=i32)
    eob = jnp.minimum(jnp.sum((blks[:, None] >= blk_end[None, :])
                              .astype(i32), axis=1), _E - 1)
    eob = jnp.concatenate([eob, blk_end[-1:]])
    pos = dst.reshape(_N, _K)
    p0 = pos[:, 0]
    p1 = pos[:, 1]
    wpad = jnp.zeros((_N, _E), jnp.float32)
    wpad = wpad.at[:, 0].set(w1v).at[:, 1].set(w2v)

    ys = _ffn(xf, w1b, b1.reshape(_E, 1, _H), w2b, b2.reshape(_E, 1, _D),
              eob, tok_sorted)
    out = _combine(ys, wpad, p0, p1)
    return out.reshape(_B, _S, _D)
